# Initial kernel scaffold; baseline (speedup 1.0000x reference)
import jax
import jax.numpy as jnp
from jax.experimental import pallas as pl


def kernel(x, edge_index, W_down, b_down, p_attn, W_up, b_up):
    raise NotImplementedError("write your pallas kernel here")


# speedup vs baseline: 2.3565x; 2.3565x over previous
import functools
import math

import jax
import jax.numpy as jnp
from jax.experimental import pallas as pl

_DEPTH = 3


def _round_up(n, m=256):
    return ((n + m - 1) // m) * m


def _mm_kernel(a_ref, b_ref, o_ref, *, trans_a):
    @pl.when(pl.program_id(2) == 0)
    def _init():
        o_ref[...] = jnp.zeros_like(o_ref)

    a = a_ref[...]
    b = b_ref[...]
    if trans_a:
        acc = jax.lax.dot_general(
            a, b, (((0,), (0,)), ((), ())), preferred_element_type=jnp.float32
        )
    else:
        acc = jnp.dot(a, b, preferred_element_type=jnp.float32)
    o_ref[...] += acc


def _pick(n):
    for cand in (1024, 512, 256):
        if n % cand == 0:
            return cand
    return n


def _mm(a, b, trans_a=False):
    if trans_a:
        k_dim, m_dim = a.shape
    else:
        m_dim, k_dim = a.shape
    n_dim = b.shape[1]
    bm, bn, bk = _pick(m_dim), _pick(n_dim), _pick(k_dim)
    grid = (m_dim // bm, n_dim // bn, k_dim // bk)
    if trans_a:
        a_spec = pl.BlockSpec((bk, bm), lambda i, j, k: (k, i))
    else:
        a_spec = pl.BlockSpec((bm, bk), lambda i, j, k: (i, k))
    b_spec = pl.BlockSpec((bk, bn), lambda i, j, k: (k, j))
    o_spec = pl.BlockSpec((bm, bn), lambda i, j, k: (i, j))
    return pl.pallas_call(
        functools.partial(_mm_kernel, trans_a=trans_a),
        grid=grid,
        in_specs=[a_spec, b_spec],
        out_specs=o_spec,
        out_shape=jax.ShapeDtypeStruct((m_dim, n_dim), jnp.float32),
    )(a, b)


def _colsum_kernel(a_ref, o_ref):
    @pl.when(pl.program_id(1) == 0)
    def _init():
        o_ref[...] = jnp.zeros_like(o_ref)

    o_ref[...] += jnp.sum(a_ref[...], axis=0, keepdims=True)


def _colsum(a):
    n = a.shape[0]
    bm = _pick(n)
    grid = (n // bm, n // bm)
    out = pl.pallas_call(
        _colsum_kernel,
        grid=grid,
        in_specs=[pl.BlockSpec((bm, bm), lambda j, r: (r, j))],
        out_specs=pl.BlockSpec((1, bm), lambda j, r: (0, j)),
        out_shape=jax.ShapeDtypeStruct((1, n), jnp.float32),
    )(a)
    return out[0]


def _gcn(A, diagA, x, W, b):
    y = _mm(x, W)
    deg = _colsum(A) - diagA + 2.0
    dis = jax.lax.rsqrt(deg)
    y2 = y * dis[:, None]
    z = _mm(A, y2, trans_a=True)
    z = z + (2.0 - diagA)[:, None] * y2
    return dis[:, None] * z + b[None, :]


def kernel(x, edge_index, W_down, b_down, p_attn, W_up, b_up):
    n = x.shape[0]
    n_pad = _round_up(n)
    A = jnp.zeros((n_pad, n_pad), jnp.float32)
    A = A.at[edge_index[0], edge_index[1]].add(1.0)
    diagA = jnp.diagonal(A)
    x = jnp.pad(x.astype(jnp.float32), ((0, n_pad - n), (0, 0)))

    x = jax.nn.relu(_gcn(A, diagA, x, W_down[0], b_down[0]))
    xs = [x]
    As = [(A, diagA)]
    perms = []

    n_real, n_cur = n, n_pad
    for i in range(_DEPTH):
        p = p_attn[i]
        score = jnp.tanh((x @ p) / jnp.linalg.norm(p))
        score = jnp.where(jnp.arange(n_cur) < n_real, score, -2.0)
        k = int(math.ceil(0.5 * n_real))
        k_pad = _round_up(k)
        vals, perm = jax.lax.top_k(score, k)
        perm_ext = jnp.concatenate(
            [perm, jnp.full((k_pad - k,), n_cur - 1, dtype=perm.dtype)]
        )
        vals_ext = jnp.concatenate([vals, jnp.zeros((k_pad - k,), vals.dtype)])

        d = 1.0 - diagA
        P = jnp.take(A, perm_ext, axis=0)
        Q = jnp.take(A, perm_ext, axis=1)
        Asub = jnp.take(P, perm_ext, axis=1)
        dp = jnp.take(d, perm_ext)
        M = _mm(P, Q)
        A_new = M + Asub * (dp[:, None] + dp[None, :])
        ar = jnp.arange(k_pad)
        A_new = A_new.at[ar, ar].set(0.0)

        x = jnp.take(x, perm_ext, axis=0) * vals_ext[:, None]
        A, diagA = A_new, jnp.zeros((k_pad,), jnp.float32)
        n_real, n_cur = k, k_pad

        x = jax.nn.relu(_gcn(A, diagA, x, W_down[i + 1], b_down[i + 1]))
        if i < _DEPTH - 1:
            xs.append(x)
            As.append((A, diagA))
        perms.append(perm_ext)

    for i in range(_DEPTH):
        j = _DEPTH - 1 - i
        res = xs[j]
        Aj, dAj = As[j]
        perm_ext = perms[j]
        up = jnp.zeros_like(res).at[perm_ext].set(x)
        x = res + up
        x = _gcn(Aj, dAj, x, W_up[i], b_up[i])
        if i < _DEPTH - 1:
            x = jax.nn.relu(x)
    return x[:n]
